# baseline (speedup 1.0000x reference)
import functools
import jax
import jax.numpy as jnp
from jax import lax
from jax.experimental import pallas as pl
from jax.experimental.pallas import tpu as pltpu
from jax.experimental.pallas import tpu_sc as plsc

N_NODES = 10000
N_EDGES = 320000
D = 128
NPAD = N_NODES
R = 1000
GRID = NPAD // R
NW = 32
EPW = N_EDGES // NW
CH = 50
NCH_W = EPW // CH
U = 6
NT = 32
NB = 6
G = 4
NI = 6
AI = 5
RPT = 640


def _sc_body(m_hbm, src_hbm, dst_hbm, zero_hbm, out_hbm, *scr):
    sidx = list(scr[0:NI])
    didx = list(scr[NI:2 * NI])
    rows = list(scr[2 * NI:2 * NI + NB])
    acc = scr[2 * NI + NB]
    s_sems = list(scr[2 * NI + NB + 1:3 * NI + NB + 1])
    d_sems = list(scr[3 * NI + NB + 1:4 * NI + NB + 1])
    gsems = list(scr[4 * NI + NB + 1:4 * NI + 2 * NB + 1])
    cid = lax.axis_index("c")
    sid = lax.axis_index("s")
    wid = sid * 2 + cid
    crow = wid * NCH_W

    def start_idx(j, bi):
        pltpu.async_copy(src_hbm.at[crow + j], sidx[bi], s_sems[bi])
        pltpu.async_copy(dst_hbm.at[crow + j], didx[bi], d_sems[bi])

    def wait_idx_s(j, bi):
        pltpu.make_async_copy(src_hbm.at[crow + j], sidx[bi],
                              s_sems[bi]).wait()

    def wait_idx_d(j, bi):
        pltpu.make_async_copy(dst_hbm.at[crow + j], didx[bi],
                              d_sems[bi]).wait()

    def start_gather(bi, b):
        pltpu.async_copy(m_hbm.at[sidx[bi]], rows[b], gsems[b])

    def wait_gather(bi, b):
        pltpu.make_async_copy(m_hbm.at[sidx[bi]], rows[b], gsems[b]).wait()

    def scatter(bi, b):
        pltpu.sync_copy(rows[b], acc.at[didx[bi]], add=True)

    for k in range(AI):
        start_idx(k, k)
    for k in range(G):
        wait_idx_s(k, k)
        start_gather(k, k)

    r0 = sid * RPT

    @pl.when(sid < 15)
    def _():
        pltpu.sync_copy(zero_hbm, acc.at[pl.ds(r0, RPT)])

    @pl.when(sid == 15)
    def _():
        pltpu.sync_copy(zero_hbm.at[pl.ds(0, NPAD - 15 * RPT)],
                        acc.at[pl.ds(15 * RPT, NPAD - 15 * RPT)])

    plsc.subcore_barrier()

    def step(j, u, gather_next, idx_next):
        bi = u % NI
        b = u % NB
        wait_gather(bi, b)
        wait_idx_d(j, bi)
        scatter(bi, b)
        if gather_next:
            ng = (u + G) % NI
            wait_idx_s(j + G, ng)
            start_gather(ng, (u + G) % NB)
        if idx_next:
            start_idx(j + AI, (u + AI) % NI)

    def body(t, carry):
        for u in range(U):
            step(t * U + u, u, True, True)
        return carry

    lax.fori_loop(0, NT, body, 0)
    for j in range(NT * U, NCH_W):
        step(j, j % U, j + G < NCH_W, j + AI < NCH_W)
    plsc.subcore_barrier()

    @pl.when(sid < 15)
    def _():
        pltpu.sync_copy(acc.at[pl.ds(r0, RPT)],
                        out_hbm.at[pl.ds(cid * NPAD + r0, RPT)])

    @pl.when(sid == 15)
    def _():
        pltpu.sync_copy(
            acc.at[pl.ds(15 * RPT, NPAD - 15 * RPT)],
            out_hbm.at[pl.ds(cid * NPAD + 15 * RPT, NPAD - 15 * RPT)])


def _make_sc_scatter():
    mesh = plsc.VectorSubcoreMesh(core_axis_name="c", subcore_axis_name="s")
    return functools.partial(
        pl.kernel,
        mesh=mesh,
        out_type=jax.ShapeDtypeStruct((2 * NPAD, D), jnp.float32),
        scratch_types=(
            [pltpu.VMEM((CH,), jnp.int32) for _ in range(2 * NI)]
            + [pltpu.VMEM((CH, D), jnp.float32) for _ in range(NB)]
            + [pltpu.VMEM_SHARED((NPAD, D), jnp.float32)]
            + [pltpu.SemaphoreType.DMA for _ in range(2 * NI + NB)]
        ),
    )(_sc_body)


_sc_scatter = _make_sc_scatter()


def _dot_t(a, w):
    return lax.dot_general(a, w, (((1,), (1,)), ((), ())),
                           preferred_element_type=jnp.float32)


def _k_inproj(x_ref, wt_ref, b_ref, g_ref, h_ref, m_ref):
    h = _dot_t(x_ref[...], wt_ref[...]) + b_ref[...]
    h_ref[...] = h
    m_ref[...] = jnp.dot(h, g_ref[...], preferred_element_type=jnp.float32)


def _gru(p0, p1, h, wih, whh, bih, bhh):
    agg = p0 + p1
    gi = _dot_t(agg, wih) + bih
    gh = _dot_t(h, whh) + bhh
    r = jax.nn.sigmoid(gi[:, 0:D] + gh[:, 0:D])
    z = jax.nn.sigmoid(gi[:, D:2 * D] + gh[:, D:2 * D])
    n = jnp.tanh(gi[:, 2 * D:3 * D] + r * gh[:, 2 * D:3 * D])
    return (1.0 - z) * n + z * h


def _k_gru_next(p0_ref, p1_ref, h_ref, wih_ref, whh_ref, bih_ref, bhh_ref,
                g_ref, hout_ref, mout_ref):
    h2 = _gru(p0_ref[...], p1_ref[...], h_ref[...], wih_ref[...],
              whh_ref[...], bih_ref[...], bhh_ref[...])
    hout_ref[...] = h2
    mout_ref[...] = jnp.dot(h2, g_ref[...], preferred_element_type=jnp.float32)


def _k_gru_final(p0_ref, p1_ref, h_ref, wih_ref, whh_ref, bih_ref, bhh_ref,
                 owt_ref, ob_ref, out_ref):
    h2 = _gru(p0_ref[...], p1_ref[...], h_ref[...], wih_ref[...],
              whh_ref[...], bih_ref[...], bhh_ref[...])
    h2 = jnp.maximum(h2, 0.0)
    o = _dot_t(h2, owt_ref[...]) + ob_ref[...]
    mx = jnp.max(o, axis=1, keepdims=True)
    lse = jnp.log(jnp.sum(jnp.exp(o - mx), axis=1, keepdims=True)) + mx
    out_ref[...] = o - lse


def _row_spec(off=0):
    return pl.BlockSpec((R, D), lambda i, off=off: (i + off, 0))


def _full_spec(shape):
    return pl.BlockSpec(shape, lambda i: tuple(0 for _ in shape))


def _call_inproj(xp, in_wt, in_b2, g0):
    return pl.pallas_call(
        _k_inproj,
        grid=(GRID,),
        in_specs=[_row_spec(), _full_spec((D, D)), _full_spec((1, D)),
                  _full_spec((D, D))],

        out_specs=[_row_spec(), _row_spec()],
        out_shape=[jax.ShapeDtypeStruct((NPAD, D), jnp.float32),
                   jax.ShapeDtypeStruct((NPAD, D), jnp.float32)],
    )(xp, in_wt, in_b2, g0)


def _call_gru_next(part, h, wih, whh, bih, bhh, g):
    return pl.pallas_call(
        _k_gru_next,
        grid=(GRID,),
        in_specs=[_row_spec(), _row_spec(GRID), _row_spec(),
                  _full_spec((3 * D, D)), _full_spec((3 * D, D)),
                  _full_spec((1, 3 * D)), _full_spec((1, 3 * D)),
                  _full_spec((D, D))],
        out_specs=[_row_spec(), _row_spec()],
        out_shape=[jax.ShapeDtypeStruct((NPAD, D), jnp.float32),
                   jax.ShapeDtypeStruct((NPAD, D), jnp.float32)],
    )(part, part, h, wih, whh, bih, bhh, g)


def _call_gru_final(part, h, wih, whh, bih, bhh, owt, ob2):
    return pl.pallas_call(
        _k_gru_final,
        grid=(GRID,),
        in_specs=[_row_spec(), _row_spec(GRID), _row_spec(),
                  _full_spec((3 * D, D)), _full_spec((3 * D, D)),
                  _full_spec((1, 3 * D)), _full_spec((1, 3 * D)),
                  _full_spec((D, D)), _full_spec((1, D))],
        out_specs=_row_spec(),
        out_shape=jax.ShapeDtypeStruct((NPAD, D), jnp.float32),
    )(part, part, h, wih, whh, bih, bhh, owt, ob2)


def kernel(x, edge_index, in_W, in_b, gg_weight, W_ih, W_hh, b_ih, b_hh,
           out_W, out_b):
    src2 = edge_index[0].astype(jnp.int32).reshape(NW * NCH_W, CH)
    dst2 = edge_index[1].astype(jnp.int32).reshape(NW * NCH_W, CH)
    xp = x
    in_wt = in_W
    wih = W_ih
    whh = W_hh
    owt = out_W
    in_b2 = in_b.reshape(1, D)
    bih2 = b_ih.reshape(1, 3 * D)
    bhh2 = b_hh.reshape(1, 3 * D)
    ob2 = out_b.reshape(1, D)
    zeros = jnp.zeros((RPT, D), jnp.float32)

    h, m = _call_inproj(xp, in_wt, in_b2, gg_weight[0])
    for i in range(3):
        part = _sc_scatter(m, src2, dst2, zeros)
        if i < 2:
            h, m = _call_gru_next(part, h, wih, whh, bih2, bhh2,
                                  gg_weight[i + 1])
        else:
            out = _call_gru_final(part, h, wih, whh, bih2, bhh2, owt, ob2)
    return out


# speedup vs baseline: 1.1030x; 1.1030x over previous
import functools
import jax
import jax.numpy as jnp
from jax import lax
from jax.experimental import pallas as pl
from jax.experimental.pallas import tpu as pltpu
from jax.experimental.pallas import tpu_sc as plsc

N_NODES = 10000
N_EDGES = 320000
D = 128
NPAD = N_NODES
R = 1000
GRID = NPAD // R
NW = 32
EPW = N_EDGES // NW
CH = 125
NCH_W = EPW // CH
U = 12
NT = 6
NB = 3
G = 2
NI = 4
AI = 3
RPT = 640


def _sc_body(m_hbm, src_hbm, dst_hbm, zero_hbm, out_hbm, *scr):
    sidx = list(scr[0:NI])
    didx = list(scr[NI:2 * NI])
    rows = list(scr[2 * NI:2 * NI + NB])
    acc = scr[2 * NI + NB]
    s_sems = list(scr[2 * NI + NB + 1:3 * NI + NB + 1])
    d_sems = list(scr[3 * NI + NB + 1:4 * NI + NB + 1])
    gsems = list(scr[4 * NI + NB + 1:4 * NI + 2 * NB + 1])
    cid = lax.axis_index("c")
    sid = lax.axis_index("s")
    wid = sid * 2 + cid
    crow = wid * NCH_W

    def start_idx(j, bi):
        pltpu.async_copy(src_hbm.at[crow + j], sidx[bi], s_sems[bi])
        pltpu.async_copy(dst_hbm.at[crow + j], didx[bi], d_sems[bi])

    def wait_idx_s(j, bi):
        pltpu.make_async_copy(src_hbm.at[crow + j], sidx[bi],
                              s_sems[bi]).wait()

    def wait_idx_d(j, bi):
        pltpu.make_async_copy(dst_hbm.at[crow + j], didx[bi],
                              d_sems[bi]).wait()

    def start_gather(bi, b):
        pltpu.async_copy(m_hbm.at[sidx[bi]], rows[b], gsems[b])

    def wait_gather(bi, b):
        pltpu.make_async_copy(m_hbm.at[sidx[bi]], rows[b], gsems[b]).wait()

    def scatter(bi, b):
        pltpu.sync_copy(rows[b], acc.at[didx[bi]], add=True)

    for k in range(AI):
        start_idx(k, k)
    for k in range(G):
        wait_idx_s(k, k)
        start_gather(k, k % NB)

    r0 = sid * RPT

    @pl.when(sid < 15)
    def _():
        pltpu.sync_copy(zero_hbm, acc.at[pl.ds(r0, RPT)])

    @pl.when(sid == 15)
    def _():
        pltpu.sync_copy(zero_hbm.at[pl.ds(0, NPAD - 15 * RPT)],
                        acc.at[pl.ds(15 * RPT, NPAD - 15 * RPT)])

    plsc.subcore_barrier()

    def step(j, u, gather_next, idx_next):
        bi = u % NI
        b = u % NB
        wait_gather(bi, b)
        wait_idx_d(j, bi)
        scatter(bi, b)
        if gather_next:
            ng = (u + G) % NI
            wait_idx_s(j + G, ng)
            start_gather(ng, (u + G) % NB)
        if idx_next:
            start_idx(j + AI, (u + AI) % NI)

    def body(t, carry):
        for u in range(U):
            step(t * U + u, u, True, True)
        return carry

    lax.fori_loop(0, NT, body, 0)
    for j in range(NT * U, NCH_W):
        step(j, j % U, j + G < NCH_W, j + AI < NCH_W)
    plsc.subcore_barrier()

    @pl.when(sid < 15)
    def _():
        pltpu.sync_copy(acc.at[pl.ds(r0, RPT)],
                        out_hbm.at[pl.ds(cid * NPAD + r0, RPT)])

    @pl.when(sid == 15)
    def _():
        pltpu.sync_copy(
            acc.at[pl.ds(15 * RPT, NPAD - 15 * RPT)],
            out_hbm.at[pl.ds(cid * NPAD + 15 * RPT, NPAD - 15 * RPT)])


def _make_sc_scatter():
    mesh = plsc.VectorSubcoreMesh(core_axis_name="c", subcore_axis_name="s")
    return functools.partial(
        pl.kernel,
        mesh=mesh,
        out_type=jax.ShapeDtypeStruct((2 * NPAD, D), jnp.float32),
        scratch_types=(
            [pltpu.VMEM((CH,), jnp.int32) for _ in range(2 * NI)]
            + [pltpu.VMEM((CH, D), jnp.float32) for _ in range(NB)]
            + [pltpu.VMEM_SHARED((NPAD, D), jnp.float32)]
            + [pltpu.SemaphoreType.DMA for _ in range(2 * NI + NB)]
        ),
    )(_sc_body)


_sc_scatter = _make_sc_scatter()


def _dot_t(a, w):
    return lax.dot_general(a, w, (((1,), (1,)), ((), ())),
                           preferred_element_type=jnp.float32)


def _k_inproj(x_ref, wt_ref, b_ref, g_ref, h_ref, m_ref):
    h = _dot_t(x_ref[...], wt_ref[...]) + b_ref[...]
    h_ref[...] = h
    m_ref[...] = jnp.dot(h, g_ref[...], preferred_element_type=jnp.float32)


def _gru(p0, p1, h, wih, whh, bih, bhh):
    agg = p0 + p1
    gi = _dot_t(agg, wih) + bih
    gh = _dot_t(h, whh) + bhh
    r = jax.nn.sigmoid(gi[:, 0:D] + gh[:, 0:D])
    z = jax.nn.sigmoid(gi[:, D:2 * D] + gh[:, D:2 * D])
    n = jnp.tanh(gi[:, 2 * D:3 * D] + r * gh[:, 2 * D:3 * D])
    return (1.0 - z) * n + z * h


def _k_gru_next(p0_ref, p1_ref, h_ref, wih_ref, whh_ref, bih_ref, bhh_ref,
                g_ref, hout_ref, mout_ref):
    h2 = _gru(p0_ref[...], p1_ref[...], h_ref[...], wih_ref[...],
              whh_ref[...], bih_ref[...], bhh_ref[...])
    hout_ref[...] = h2
    mout_ref[...] = jnp.dot(h2, g_ref[...], preferred_element_type=jnp.float32)


def _k_gru_final(p0_ref, p1_ref, h_ref, wih_ref, whh_ref, bih_ref, bhh_ref,
                 owt_ref, ob_ref, out_ref):
    h2 = _gru(p0_ref[...], p1_ref[...], h_ref[...], wih_ref[...],
              whh_ref[...], bih_ref[...], bhh_ref[...])
    h2 = jnp.maximum(h2, 0.0)
    o = _dot_t(h2, owt_ref[...]) + ob_ref[...]
    mx = jnp.max(o, axis=1, keepdims=True)
    lse = jnp.log(jnp.sum(jnp.exp(o - mx), axis=1, keepdims=True)) + mx
    out_ref[...] = o - lse


def _row_spec(off=0):
    return pl.BlockSpec((R, D), lambda i, off=off: (i + off, 0))


def _full_spec(shape):
    return pl.BlockSpec(shape, lambda i: tuple(0 for _ in shape))


def _call_inproj(xp, in_wt, in_b2, g0):
    return pl.pallas_call(
        _k_inproj,
        grid=(GRID,),
        in_specs=[_row_spec(), _full_spec((D, D)), _full_spec((1, D)),
                  _full_spec((D, D))],

        out_specs=[_row_spec(), _row_spec()],
        out_shape=[jax.ShapeDtypeStruct((NPAD, D), jnp.float32),
                   jax.ShapeDtypeStruct((NPAD, D), jnp.float32)],
    )(xp, in_wt, in_b2, g0)


def _call_gru_next(part, h, wih, whh, bih, bhh, g):
    return pl.pallas_call(
        _k_gru_next,
        grid=(GRID,),
        in_specs=[_row_spec(), _row_spec(GRID), _row_spec(),
                  _full_spec((3 * D, D)), _full_spec((3 * D, D)),
                  _full_spec((1, 3 * D)), _full_spec((1, 3 * D)),
                  _full_spec((D, D))],
        out_specs=[_row_spec(), _row_spec()],
        out_shape=[jax.ShapeDtypeStruct((NPAD, D), jnp.float32),
                   jax.ShapeDtypeStruct((NPAD, D), jnp.float32)],
    )(part, part, h, wih, whh, bih, bhh, g)


def _call_gru_final(part, h, wih, whh, bih, bhh, owt, ob2):
    return pl.pallas_call(
        _k_gru_final,
        grid=(GRID,),
        in_specs=[_row_spec(), _row_spec(GRID), _row_spec(),
                  _full_spec((3 * D, D)), _full_spec((3 * D, D)),
                  _full_spec((1, 3 * D)), _full_spec((1, 3 * D)),
                  _full_spec((D, D)), _full_spec((1, D))],
        out_specs=_row_spec(),
        out_shape=jax.ShapeDtypeStruct((NPAD, D), jnp.float32),
    )(part, part, h, wih, whh, bih, bhh, owt, ob2)


def kernel(x, edge_index, in_W, in_b, gg_weight, W_ih, W_hh, b_ih, b_hh,
           out_W, out_b):
    src2 = edge_index[0].astype(jnp.int32).reshape(NW * NCH_W, CH)
    dst2 = edge_index[1].astype(jnp.int32).reshape(NW * NCH_W, CH)
    xp = x
    in_wt = in_W
    wih = W_ih
    whh = W_hh
    owt = out_W
    in_b2 = in_b.reshape(1, D)
    bih2 = b_ih.reshape(1, 3 * D)
    bhh2 = b_hh.reshape(1, 3 * D)
    ob2 = out_b.reshape(1, D)
    zeros = jnp.zeros((RPT, D), jnp.float32)

    h, m = _call_inproj(xp, in_wt, in_b2, gg_weight[0])
    for i in range(3):
        part = _sc_scatter(m, src2, dst2, zeros)
        if i < 2:
            h, m = _call_gru_next(part, h, wih, whh, bih2, bhh2,
                                  gg_weight[i + 1])
        else:
            out = _call_gru_final(part, h, wih, whh, bih2, bhh2, owt, ob2)
    return out


# speedup vs baseline: 1.2174x; 1.1037x over previous
import functools
import jax
import jax.numpy as jnp
from jax import lax
from jax.experimental import pallas as pl
from jax.experimental.pallas import tpu as pltpu
from jax.experimental.pallas import tpu_sc as plsc

N_NODES = 10000
N_EDGES = 320000
D = 128
NPAD = N_NODES
R = 1000
GRID = NPAD // R
NW = 32
EPW = N_EDGES // NW
CH = 80
NCH_W = EPW // CH
U = 8
NT = 15
NB = 4
NI = 8
AI = 4
RPT = 640


def _sc_body(m_hbm, src_hbm, dst_hbm, zero_hbm, out_hbm, *scr):
    sidx = list(scr[0:NI])
    didx = list(scr[NI:2 * NI])
    rows = list(scr[2 * NI:2 * NI + NB])
    acc = scr[2 * NI + NB]
    s_sems = list(scr[2 * NI + NB + 1:3 * NI + NB + 1])
    d_sems = list(scr[3 * NI + NB + 1:4 * NI + NB + 1])
    gsems = list(scr[4 * NI + NB + 1:4 * NI + 2 * NB + 1])
    cid = lax.axis_index("c")
    sid = lax.axis_index("s")
    wid = sid * 2 + cid
    crow = wid * NCH_W

    def start_idx(j, bi):
        pltpu.async_copy(src_hbm.at[crow + j], sidx[bi], s_sems[bi])
        pltpu.async_copy(dst_hbm.at[crow + j], didx[bi], d_sems[bi])

    def wait_idx_s(j, bi):
        pltpu.make_async_copy(src_hbm.at[crow + j], sidx[bi],
                              s_sems[bi]).wait()

    def wait_idx_d(j, bi):
        pltpu.make_async_copy(dst_hbm.at[crow + j], didx[bi],
                              d_sems[bi]).wait()

    def start_gather(bi, b):
        pltpu.async_copy(m_hbm.at[sidx[bi]], rows[b], gsems[b])

    def wait_gather(bi, b):
        pltpu.make_async_copy(m_hbm.at[sidx[bi]], rows[b], gsems[b]).wait()

    def scatter(bi, b):
        pltpu.sync_copy(rows[b], acc.at[didx[bi]], add=True)

    for k in range(AI):
        start_idx(k, k)
    for k in range(NB - 1):
        wait_idx_s(k, k)
        start_gather(k, k)

    r0 = sid * RPT

    @pl.when(sid < 15)
    def _():
        pltpu.sync_copy(zero_hbm, acc.at[pl.ds(r0, RPT)])

    @pl.when(sid == 15)
    def _():
        pltpu.sync_copy(zero_hbm.at[pl.ds(0, NPAD - 15 * RPT)],
                        acc.at[pl.ds(15 * RPT, NPAD - 15 * RPT)])

    plsc.subcore_barrier()

    def step(j, u, gather_next, idx_next):
        bi = u % NI
        b = u % NB
        wait_gather(bi, b)
        wait_idx_d(j, bi)
        scatter(bi, b)
        if gather_next:
            ng = (u + NB - 1) % NI
            wait_idx_s(j + NB - 1, ng)
            start_gather(ng, (u + NB - 1) % NB)
        if idx_next:
            start_idx(j + AI, (u + AI) % NI)

    def body(t, carry):
        for u in range(U):
            step(t * U + u, u, True, True)
        return carry

    lax.fori_loop(0, NT, body, 0)
    for j in range(NT * U, NCH_W):
        step(j, j % U, j + NB - 1 < NCH_W, j + AI < NCH_W)
    plsc.subcore_barrier()

    @pl.when(sid < 15)
    def _():
        pltpu.sync_copy(acc.at[pl.ds(r0, RPT)],
                        out_hbm.at[pl.ds(cid * NPAD + r0, RPT)])

    @pl.when(sid == 15)
    def _():
        pltpu.sync_copy(
            acc.at[pl.ds(15 * RPT, NPAD - 15 * RPT)],
            out_hbm.at[pl.ds(cid * NPAD + 15 * RPT, NPAD - 15 * RPT)])


def _make_sc_scatter():
    mesh = plsc.VectorSubcoreMesh(core_axis_name="c", subcore_axis_name="s")
    return functools.partial(
        pl.kernel,
        mesh=mesh,
        out_type=jax.ShapeDtypeStruct((2 * NPAD, D), jnp.float32),
        scratch_types=(
            [pltpu.VMEM((CH,), jnp.int32) for _ in range(2 * NI)]
            + [pltpu.VMEM((CH, D), jnp.float32) for _ in range(NB)]
            + [pltpu.VMEM_SHARED((NPAD, D), jnp.float32)]
            + [pltpu.SemaphoreType.DMA for _ in range(2 * NI + NB)]
        ),
    )(_sc_body)


_sc_scatter = _make_sc_scatter()


def _k_inproj(x_ref, wt_ref, b_ref, g_ref, h_ref, m_ref):
    h = jnp.dot(x_ref[...], wt_ref[...],
                preferred_element_type=jnp.float32) + b_ref[...]
    h_ref[...] = h
    m_ref[...] = jnp.dot(h, g_ref[...], preferred_element_type=jnp.float32)


def _gru(p0, p1, h, wih, whh, bih, bhh):
    agg = p0 + p1
    gi = jnp.dot(agg, wih, preferred_element_type=jnp.float32) + bih
    gh = jnp.dot(h, whh, preferred_element_type=jnp.float32) + bhh
    r = jax.nn.sigmoid(gi[:, 0:D] + gh[:, 0:D])
    z = jax.nn.sigmoid(gi[:, D:2 * D] + gh[:, D:2 * D])
    n = jnp.tanh(gi[:, 2 * D:3 * D] + r * gh[:, 2 * D:3 * D])
    return (1.0 - z) * n + z * h


def _k_gru_next(p0_ref, p1_ref, h_ref, wih_ref, whh_ref, bih_ref, bhh_ref,
                g_ref, hout_ref, mout_ref):
    h2 = _gru(p0_ref[...], p1_ref[...], h_ref[...], wih_ref[...],
              whh_ref[...], bih_ref[...], bhh_ref[...])
    hout_ref[...] = h2
    mout_ref[...] = jnp.dot(h2, g_ref[...], preferred_element_type=jnp.float32)


def _k_gru_final(p0_ref, p1_ref, h_ref, wih_ref, whh_ref, bih_ref, bhh_ref,
                 owt_ref, ob_ref, out_ref):
    h2 = _gru(p0_ref[...], p1_ref[...], h_ref[...], wih_ref[...],
              whh_ref[...], bih_ref[...], bhh_ref[...])
    h2 = jnp.maximum(h2, 0.0)
    o = jnp.dot(h2, owt_ref[...], preferred_element_type=jnp.float32) + ob_ref[...]
    mx = jnp.max(o, axis=1, keepdims=True)
    lse = jnp.log(jnp.sum(jnp.exp(o - mx), axis=1, keepdims=True)) + mx
    out_ref[...] = o - lse


def _row_spec(off=0):
    return pl.BlockSpec((R, D), lambda i, off=off: (i + off, 0))


def _full_spec(shape):
    return pl.BlockSpec(shape, lambda i: tuple(0 for _ in shape))


def _call_inproj(xp, in_wt, in_b2, g0):
    return pl.pallas_call(
        _k_inproj,
        grid=(GRID,),
        in_specs=[_row_spec(), _full_spec((D, D)), _full_spec((1, D)),
                  _full_spec((D, D))],
        out_specs=[_row_spec(), _row_spec()],
        out_shape=[jax.ShapeDtypeStruct((NPAD, D), jnp.float32),
                   jax.ShapeDtypeStruct((NPAD, D), jnp.float32)],
    )(xp, in_wt, in_b2, g0)


def _call_gru_next(part, h, wih, whh, bih, bhh, g):
    return pl.pallas_call(
        _k_gru_next,
        grid=(GRID,),
        in_specs=[_row_spec(), _row_spec(GRID), _row_spec(),
                  _full_spec((D, 3 * D)), _full_spec((D, 3 * D)),
                  _full_spec((1, 3 * D)), _full_spec((1, 3 * D)),
                  _full_spec((D, D))],
        out_specs=[_row_spec(), _row_spec()],
        out_shape=[jax.ShapeDtypeStruct((NPAD, D), jnp.float32),
                   jax.ShapeDtypeStruct((NPAD, D), jnp.float32)],
    )(part, part, h, wih, whh, bih, bhh, g)


def _call_gru_final(part, h, wih, whh, bih, bhh, owt, ob2):
    return pl.pallas_call(
        _k_gru_final,
        grid=(GRID,),
        in_specs=[_row_spec(), _row_spec(GRID), _row_spec(),
                  _full_spec((D, 3 * D)), _full_spec((D, 3 * D)),
                  _full_spec((1, 3 * D)), _full_spec((1, 3 * D)),
                  _full_spec((D, D)), _full_spec((1, D))],
        out_specs=_row_spec(),
        out_shape=jax.ShapeDtypeStruct((NPAD, D), jnp.float32),
    )(part, part, h, wih, whh, bih, bhh, owt, ob2)


def kernel(x, edge_index, in_W, in_b, gg_weight, W_ih, W_hh, b_ih, b_hh,
           out_W, out_b):
    src2 = edge_index[0].astype(jnp.int32).reshape(NW * NCH_W, CH)
    dst2 = edge_index[1].astype(jnp.int32).reshape(NW * NCH_W, CH)
    xp = x
    in_wt = in_W.T
    wih = W_ih.T
    whh = W_hh.T
    owt = out_W.T
    in_b2 = in_b.reshape(1, D)
    bih2 = b_ih.reshape(1, 3 * D)
    bhh2 = b_hh.reshape(1, 3 * D)
    ob2 = out_b.reshape(1, D)
    zeros = jnp.zeros((RPT, D), jnp.float32)

    h, m = _call_inproj(xp, in_wt, in_b2, gg_weight[0])
    for i in range(3):
        part = _sc_scatter(m, src2, dst2, zeros)
        if i < 2:
            h, m = _call_gru_next(part, h, wih, whh, bih2, bhh2,
                                  gg_weight[i + 1])
        else:
            out = _call_gru_final(part, h, wih, whh, bih2, bhh2, owt, ob2)
    return out
